# baseline (speedup 1.0000x reference)
import functools

import jax
import jax.numpy as jnp
import numpy as np
from jax.experimental import pallas as pl

N = 512
CH = 32
N_RBF = 80
RAD_HIDDEN = 32
TILE_I = 64
TILE_J = 128
_LOG2 = float(np.log(2.0))


def _ssp(x):
    return jnp.maximum(x, 0.0) + jnp.log1p(jnp.exp(-jnp.abs(x))) - _LOG2


def _dot(a, b):
    return jax.lax.dot_general(
        a, b, (((1,), (0,)), ((), ())),
        preferred_element_type=jnp.float32,
        precision=jax.lax.Precision.HIGHEST,
    )


def _pair_geometry(ri_ref, rT_ref, j0):
    rxi = ri_ref[:, 0:1]
    ryi = ri_ref[:, 1:2]
    rzi = ri_ref[:, 2:3]
    rxj = rT_ref[0:1, pl.ds(j0, TILE_J)]
    ryj = rT_ref[1:2, pl.ds(j0, TILE_J)]
    rzj = rT_ref[2:3, pl.ds(j0, TILE_J)]
    dx = rxi - rxj
    dy = ryi - ryj
    dz = rzi - rzj
    dist = jnp.sqrt(dx * dx + dy * dy + dz * dz + 1e-12)
    inv = 1.0 / (dist + 1e-8)
    vx = dx * inv
    vy = dy * inv
    vz = dz * inv
    centers = -1.0 + 0.2 * jax.lax.broadcasted_iota(
        jnp.int32, (1, 1, N_RBF), 2).astype(jnp.float32)
    rbf = jnp.exp((dist[:, :, None] - centers) ** 2 * (-5.0))
    return vx, vy, vz, rbf


def _radial_block(rbf, w1_ref, b1_ref, w2_ref, b2_ref):
    flat = rbf.reshape(TILE_I * TILE_J, N_RBF)
    h = jnp.maximum(_dot(flat, w1_ref[:, :]) + b1_ref[:, :], 0.0)
    f = _dot(h, w2_ref[:, :]) + b2_ref[:, :]
    return f.reshape(TILE_I, TILE_J, f.shape[-1])


def _finish(c0, s1x, s1y, s1z, si0t_ref, si1t_ref, b0_ref, bb1_ref,
            o0_ref, o1_ref):
    s0 = _dot(c0, si0t_ref[:, :])
    s1x = _dot(s1x, si1t_ref[:, :])
    s1y = _dot(s1y, si1t_ref[:, :])
    s1z = _dot(s1z, si1t_ref[:, :])
    o0_ref[:, :] = _ssp(s0 + b0_ref[:, :])
    nrm = jnp.sqrt(s1x * s1x + s1y * s1y + s1z * s1z + 1e-8)
    fac = _ssp(nrm + bb1_ref[:, :]) / nrm
    o1_ref[0, :, :] = s1x * fac
    o1_ref[1, :, :] = s1y * fac
    o1_ref[2, :, :] = s1z * fac


def _layer_a_kernel(ri_ref, rT_ref, t0_ref, w1_ref, b1_ref, w2_ref, b2_ref,
                    si0t_ref, si1t_ref, b0_ref, bb1_ref, o0_ref, o1_ref):
    def body(jc, acc):
        c0, c1x, c1y, c1z = acc
        j0 = jc * TILE_J
        vx, vy, vz, rbf = _pair_geometry(ri_ref, rT_ref, j0)
        f = _radial_block(rbf, w1_ref, b1_ref, w2_ref, b2_ref)
        f0 = f[:, :, 0:CH]
        f1 = f[:, :, CH:2 * CH]
        t0 = t0_ref[pl.ds(j0, TILE_J), :][None, :, :]
        g0 = f0 * t0
        g1 = f1 * t0
        c0 = c0 + jnp.sum(g0, axis=1)
        c1x = c1x + jnp.sum(g1 * vx[:, :, None], axis=1)
        c1y = c1y + jnp.sum(g1 * vy[:, :, None], axis=1)
        c1z = c1z + jnp.sum(g1 * vz[:, :, None], axis=1)
        return c0, c1x, c1y, c1z

    z = jnp.zeros((TILE_I, CH), jnp.float32)
    c0, c1x, c1y, c1z = jax.lax.fori_loop(
        0, N // TILE_J, body, (z, z, z, z))
    _finish(c0, c1x, c1y, c1z, si0t_ref, si1t_ref, b0_ref, bb1_ref,
            o0_ref, o1_ref)


def _layer_b_kernel(ri_ref, rT_ref, t0_ref, t1_ref, w1_ref, b1_ref, w2_ref,
                    b2_ref, si0t_ref, si1t_ref, b0_ref, bb1_ref,
                    o0_ref, o1_ref):
    def body(jc, acc):
        c0, c1x, c1y, c1z = acc
        j0 = jc * TILE_J
        vx, vy, vz, rbf = _pair_geometry(ri_ref, rT_ref, j0)
        f = _radial_block(rbf, w1_ref, b1_ref, w2_ref, b2_ref)
        f0a = f[:, :, 0:CH]
        f1a = f[:, :, CH:2 * CH]
        f0b = f[:, :, 2 * CH:3 * CH]
        f1b = f[:, :, 3 * CH:4 * CH]
        t0 = t0_ref[pl.ds(j0, TILE_J), :][None, :, :]
        t1x = t1_ref[0, pl.ds(j0, TILE_J), :][None, :, :]
        t1y = t1_ref[1, pl.ds(j0, TILE_J), :][None, :, :]
        t1z = t1_ref[2, pl.ds(j0, TILE_J), :][None, :, :]
        vX = vx[:, :, None]
        vY = vy[:, :, None]
        vZ = vz[:, :, None]
        dd = vX * t1x + vY * t1y + vZ * t1z
        c0 = c0 + jnp.concatenate(
            [jnp.sum(f0a * t0, axis=1), jnp.sum(f1b * dd, axis=1)], axis=-1)
        g1 = f1a * t0
        c1x = c1x + jnp.concatenate(
            [jnp.sum(g1 * vX, axis=1),
             jnp.sum(f0b * t1x, axis=1),
             jnp.sum(f1b * (vY * t1z - vZ * t1y), axis=1)], axis=-1)
        c1y = c1y + jnp.concatenate(
            [jnp.sum(g1 * vY, axis=1),
             jnp.sum(f0b * t1y, axis=1),
             jnp.sum(f1b * (vZ * t1x - vX * t1z), axis=1)], axis=-1)
        c1z = c1z + jnp.concatenate(
            [jnp.sum(g1 * vZ, axis=1),
             jnp.sum(f0b * t1z, axis=1),
             jnp.sum(f1b * (vX * t1y - vY * t1x), axis=1)], axis=-1)
        return c0, c1x, c1y, c1z

    z0 = jnp.zeros((TILE_I, 2 * CH), jnp.float32)
    z1 = jnp.zeros((TILE_I, 3 * CH), jnp.float32)
    c0, c1x, c1y, c1z = jax.lax.fori_loop(
        0, N // TILE_J, body, (z0, z1, z1, z1))
    _finish(c0, c1x, c1y, c1z, si0t_ref, si1t_ref, b0_ref, bb1_ref,
            o0_ref, o1_ref)


def _pack_radials(p, n_rad):
    rads = []
    for i in range(n_rad):
        rads.append(p['rad'][i]['f0'])
        rads.append(p['rad'][i]['f1'])
    k = len(rads)
    w1 = jnp.concatenate([r['w1'] for r in rads], axis=1)
    b1 = jnp.concatenate([r['b1'] for r in rads])[None, :]
    w2 = jax.scipy.linalg.block_diag(*[r['w2'] for r in rads])
    b2 = jnp.concatenate([r['b2'] for r in rads])[None, :]
    return (w1, b1, w2, b2, p['si0'].T, p['si1'].T,
            p['b0'][None, :], p['b1'][None, :])


def _run_layer(kernel_fn, r, rT, inputs, packed):
    grid = (N // TILE_I,)
    full = lambda shape: pl.BlockSpec(shape, lambda i: (0,) * len(shape))
    in_specs = [pl.BlockSpec((TILE_I, 3), lambda i: (i, 0)), full((3, N))]
    in_specs += [full(x.shape) for x in inputs]
    in_specs += [full(x.shape) for x in packed]
    return pl.pallas_call(
        kernel_fn,
        grid=grid,
        in_specs=in_specs,
        out_specs=[pl.BlockSpec((TILE_I, CH), lambda i: (i, 0)),
                   pl.BlockSpec((3, TILE_I, CH), lambda i: (0, i, 0))],
        out_shape=[jax.ShapeDtypeStruct((N, CH), jnp.float32),
                   jax.ShapeDtypeStruct((3, N, CH), jnp.float32)],
    )(r, rT, *inputs, *packed)


@jax.jit
def _forward(r, z, params):
    rT = r.T
    emb = params['w_emb'].T[z]
    packed1 = _pack_radials(params['conv1'], 1)
    o0, o1 = _run_layer(_layer_a_kernel, r, rT, [emb], packed1)
    for name in ('conv2', 'conv3', 'conv4'):
        packed = _pack_radials(params[name], 2)
        o0, o1 = _run_layer(_layer_b_kernel, r, rT, [o0, o1], packed)
    return jnp.sum(o0)


def kernel(r, z, params):
    return _forward(r, z, params)
